# baseline (speedup 1.0000x reference)
import functools

import jax
import jax.numpy as jnp
from jax import lax
from jax.experimental import pallas as pl
from jax.experimental.pallas import tpu as pltpu
from jax.experimental.pallas import tpu_sc as plsc

N_FIELDS = 26
D = 64
VOCAB = 100000
LANES = 16


@functools.lru_cache(maxsize=None)
def _build(batch: int):
    info = plsc.get_sparse_core_info()
    nc, ns = info.num_cores, info.num_subcores
    nw = nc * ns
    total = batch * N_FIELDS
    per_w = total // nw
    assert per_w * nw == total
    ch = 104
    n_ch = per_w // ch
    assert n_ch * ch == per_w

    mesh = plsc.VectorSubcoreMesh(core_axis_name="c", subcore_axis_name="s")

    @functools.partial(
        pl.kernel,
        mesh=mesh,
        compiler_params=pltpu.CompilerParams(use_tc_tiling_on_sc=False),
        out_type=jax.ShapeDtypeStruct((total, D), jnp.float32),
        scratch_types=[
            pltpu.VMEM((n_ch, ch), jnp.int32),
            pltpu.VMEM((2, ch, D), jnp.float32),
            pltpu.VMEM((N_FIELDS, D), jnp.float32),
            pltpu.SemaphoreType.DMA,
            pltpu.SemaphoreType.DMA,
            pltpu.SemaphoreType.DMA,
            pltpu.SemaphoreType.DMA,
            pltpu.SemaphoreType.DMA,
        ],
    )
    def gather_kernel(idx_hbm, tab_hbm, bias_hbm, out_hbm,
                      idx_v, rows_v, bias_v, sem0, g0, g1, s0, s1):
        wid = lax.axis_index("s") * nc + lax.axis_index("c")
        base = wid * per_w
        pltpu.async_copy(idx_hbm.at[wid], idx_v, sem0).wait()
        pltpu.async_copy(bias_hbm, bias_v, sem0).wait()
        gsem = (g0, g1)
        ssem = (s0, s1)

        def add_bias(b):
            for r0 in range(0, ch, N_FIELDS):
                for f in range(N_FIELDS):
                    r = r0 + f
                    for c0 in range(0, D, LANES):
                        rows_v[b, r, pl.ds(c0, LANES)] = (
                            rows_v[b, r, pl.ds(c0, LANES)]
                            + bias_v[f, pl.ds(c0, LANES)]
                        )

        for b in range(2):
            pltpu.async_copy(tab_hbm.at[idx_v.at[b]], rows_v.at[b], gsem[b])

        def step(j, _):
            for b in range(2):
                jj = j + b
                pltpu.make_async_copy(tab_hbm.at[idx_v.at[jj]],
                                      rows_v.at[b], gsem[b]).wait()
                add_bias(b)
                out_slice = out_hbm.at[pl.ds(base + jj * ch, ch)]
                pltpu.async_copy(rows_v.at[b], out_slice, ssem[b])

                @pl.when(jj + 2 < n_ch)
                def _():
                    pltpu.make_async_copy(rows_v.at[b], out_slice,
                                          ssem[b]).wait()
                    pltpu.async_copy(tab_hbm.at[idx_v.at[jj + 2]],
                                     rows_v.at[b], gsem[b])
            return ()

        lax.fori_loop(0, n_ch // 2, lambda i, c: step(i * 2, c), (),
                      unroll=False)
        for b in range(2):
            last = out_hbm.at[pl.ds(base + (n_ch - 2 + b) * ch, ch)]
            pltpu.make_async_copy(rows_v.at[b], last, ssem[b]).wait()

    return gather_kernel, nw, n_ch, ch


def kernel(x, tables, bias):
    batch = x.shape[0]
    gather_kernel, nw, n_ch, ch = _build(batch)
    offs = jnp.arange(N_FIELDS, dtype=jnp.int32) * VOCAB
    idx = (x.astype(jnp.int32) + offs[None, :]).reshape(nw, n_ch, ch)
    tab = tables.reshape(N_FIELDS * VOCAB, D)
    out = gather_kernel(idx, tab, bias)
    return out.reshape(batch, N_FIELDS, D)


# speedup vs baseline: 4.1080x; 4.1080x over previous
import functools

import jax
import jax.numpy as jnp
from jax import lax
from jax.experimental import pallas as pl
from jax.experimental.pallas import tpu as pltpu
from jax.experimental.pallas import tpu_sc as plsc

N_FIELDS = 26
D = 64
VOCAB = 100000
LANES = 16


@functools.lru_cache(maxsize=None)
def _build(batch: int):
    info = plsc.get_sparse_core_info()
    nc, ns = info.num_cores, info.num_subcores
    nw = nc * ns
    units = N_FIELDS * D
    per_w = units // nw
    assert per_w * nw == units
    n_vec = batch // LANES
    assert n_vec * LANES == batch

    mesh = plsc.VectorSubcoreMesh(core_axis_name="c", subcore_axis_name="s")

    @functools.partial(
        pl.kernel,
        mesh=mesh,
        compiler_params=pltpu.CompilerParams(needs_layout_passes=False),
        out_type=jax.ShapeDtypeStruct((N_FIELDS, D, batch), jnp.float32),
        scratch_types=[
            pltpu.VMEM((VOCAB,), jnp.float32),
            pltpu.VMEM((batch,), jnp.int32),
            pltpu.VMEM((batch,), jnp.float32),
            pltpu.VMEM((batch,), jnp.float32),
            pltpu.VMEM((LANES,), jnp.float32),
            pltpu.SemaphoreType.DMA,
            pltpu.SemaphoreType.DMA,
            pltpu.SemaphoreType.DMA,
            pltpu.SemaphoreType.DMA,
            pltpu.SemaphoreType.DMA,
        ],
    )
    def lookup_kernel(tab_t, x_t, bias_s, out_t,
                      row_v, xcol_v, orow_v0, orow_v1, bias_v,
                      sem, g0, g1, s0, s1):
        wid = lax.axis_index("s") * nc + lax.axis_index("c")
        u0 = wid * per_w
        orows = (orow_v0, orow_v1)
        ssem = (s0, s1)

        pltpu.async_copy(tab_t.at[u0 // D, u0 % D], row_v, g0)

        def unit(i, _):
            for b in range(2):
                u = u0 + i + b
                f = u // D
                d = u % D
                pltpu.async_copy(x_t.at[f], xcol_v, sem).wait()
                pltpu.async_copy(bias_s.at[f, d], bias_v, sem).wait()
                bvec = bias_v[...]
                pltpu.make_async_copy(tab_t.at[f, d], row_v, g0).wait()

                @pl.when(i + b >= 2)
                def _():
                    pltpu.make_async_copy(orows[b], out_t.at[f, d],
                                          ssem[b]).wait()

                def gather16(k, _):
                    iv = xcol_v[pl.ds(k * LANES, LANES)]
                    vals = plsc.load_gather(row_v, [iv])
                    orows[b][pl.ds(k * LANES, LANES)] = vals + bvec
                    return ()

                lax.fori_loop(0, n_vec, gather16, (), unroll=8)

                pltpu.async_copy(orows[b], out_t.at[f, d], ssem[b])

                @pl.when(i + b + 1 < per_w)
                def _():
                    un = u + 1
                    pltpu.async_copy(tab_t.at[un // D, un % D], row_v, g0)
            return ()

        lax.fori_loop(0, per_w // 2, lambda i, c: unit(i * 2, c), ())
        for b in range(2):
            ul = u0 + per_w - 2 + b
            pltpu.make_async_copy(orows[b], out_t.at[ul // D, ul % D],
                                  ssem[b]).wait()

    return lookup_kernel, nw


def kernel(x, tables, bias):
    batch = x.shape[0]
    lookup_kernel, _ = _build(batch)
    tab_t = jnp.swapaxes(tables, 1, 2)
    x_t = jnp.swapaxes(x, 0, 1).astype(jnp.int32)
    bias_s = jnp.broadcast_to(bias[:, :, None], (N_FIELDS, D, LANES))
    out_t = lookup_kernel(tab_t, x_t, bias_s)
    return jnp.transpose(out_t, (2, 0, 1))


# speedup vs baseline: 4.1264x; 1.0045x over previous
import functools

import jax
import jax.numpy as jnp
from jax import lax
from jax.experimental import pallas as pl
from jax.experimental.pallas import tpu as pltpu
from jax.experimental.pallas import tpu_sc as plsc

N_FIELDS = 26
D = 64
VOCAB = 100000
LANES = 16


@functools.lru_cache(maxsize=None)
def _build(batch: int):
    info = plsc.get_sparse_core_info()
    nc, ns = info.num_cores, info.num_subcores
    nw = nc * ns
    units = N_FIELDS * D
    per_w = units // nw
    assert per_w * nw == units
    n_vec = batch // LANES
    assert n_vec * LANES == batch

    mesh = plsc.VectorSubcoreMesh(core_axis_name="c", subcore_axis_name="s")

    @functools.partial(
        pl.kernel,
        mesh=mesh,
        compiler_params=pltpu.CompilerParams(needs_layout_passes=False),
        out_type=jax.ShapeDtypeStruct((N_FIELDS, D, batch), jnp.float32),
        scratch_types=[
            pltpu.VMEM((VOCAB,), jnp.float32),
            pltpu.VMEM((batch,), jnp.int32),
            pltpu.VMEM((batch,), jnp.float32),
            pltpu.VMEM((batch,), jnp.float32),
            pltpu.VMEM((D, LANES), jnp.float32),
            pltpu.SemaphoreType.DMA,
            pltpu.SemaphoreType.DMA,
            pltpu.SemaphoreType.DMA,
            pltpu.SemaphoreType.DMA,
            pltpu.SemaphoreType.DMA,
        ],
    )
    def lookup_kernel(tab_t, x_t, bias_s, out_t,
                      row_v, xcol_v, orow_v0, orow_v1, bias_v,
                      sem, g0, g1, s0, s1):
        wid = lax.axis_index("s") * nc + lax.axis_index("c")
        u0 = wid * per_w
        orows = (orow_v0, orow_v1)
        ssem = (s0, s1)

        f0 = u0 // D
        pltpu.async_copy(x_t.at[f0], xcol_v, sem).wait()
        pltpu.async_copy(bias_s.at[f0], bias_v, sem).wait()
        pltpu.async_copy(tab_t.at[u0 // D, u0 % D], row_v, g0)

        def unit(i, _):
            for b in range(2):
                u = u0 + i + b
                f = u // D
                d = u % D
                @pl.when(jnp.logical_and(i + b > 0, d == 0))
                def _():
                    pltpu.async_copy(x_t.at[f], xcol_v, sem).wait()
                    pltpu.async_copy(bias_s.at[f], bias_v, sem).wait()

                bvec = bias_v[d]
                pltpu.make_async_copy(tab_t.at[f, d], row_v, g0).wait()

                @pl.when(i + b >= 2)
                def _():
                    pltpu.make_async_copy(orows[b], out_t.at[f, d],
                                          ssem[b]).wait()

                def gather16(k, _):
                    iv = xcol_v[pl.ds(k * LANES, LANES)]
                    vals = plsc.load_gather(row_v, [iv])
                    orows[b][pl.ds(k * LANES, LANES)] = vals + bvec
                    return ()

                lax.fori_loop(0, n_vec, gather16, (), unroll=8)

                pltpu.async_copy(orows[b], out_t.at[f, d], ssem[b])

                @pl.when(i + b + 1 < per_w)
                def _():
                    un = u + 1
                    pltpu.async_copy(tab_t.at[un // D, un % D], row_v, g0)
            return ()

        lax.fori_loop(0, per_w // 2, lambda i, c: unit(i * 2, c), ())
        for b in range(2):
            ul = u0 + per_w - 2 + b
            pltpu.make_async_copy(orows[b], out_t.at[ul // D, ul % D],
                                  ssem[b]).wait()

    return lookup_kernel, nw


def kernel(x, tables, bias):
    batch = x.shape[0]
    lookup_kernel, _ = _build(batch)
    tab_t = jnp.swapaxes(tables, 1, 2)
    x_t = jnp.swapaxes(x, 0, 1).astype(jnp.int32)
    bias_s = jnp.broadcast_to(bias[:, :, None], (N_FIELDS, D, LANES))
    out_t = lookup_kernel(tab_t, x_t, bias_s)
    return jnp.transpose(out_t, (2, 0, 1))


# speedup vs baseline: 5.6969x; 1.3806x over previous
import functools

import jax
import jax.numpy as jnp
from jax import lax
from jax.experimental import pallas as pl
from jax.experimental.pallas import tpu as pltpu
from jax.experimental.pallas import tpu_sc as plsc

N_FIELDS = 26
D = 64
VOCAB = 100000
LANES = 16


@functools.lru_cache(maxsize=None)
def _build(batch: int):
    info = plsc.get_sparse_core_info()
    nc, ns = info.num_cores, info.num_subcores
    nw = nc * ns
    units = N_FIELDS * D
    per_w = units // nw
    assert per_w * nw == units
    n_vec = batch // LANES
    assert n_vec * LANES == batch

    mesh = plsc.VectorSubcoreMesh(core_axis_name="c", subcore_axis_name="s")

    @functools.partial(
        pl.kernel,
        mesh=mesh,
        compiler_params=pltpu.CompilerParams(needs_layout_passes=False),
        out_type=jax.ShapeDtypeStruct((N_FIELDS, D, batch), jnp.float32),
        scratch_types=[
            pltpu.VMEM((VOCAB,), jnp.float32),
            pltpu.VMEM((batch,), jnp.int32),
            pltpu.VMEM((batch,), jnp.float32),
            pltpu.VMEM((batch,), jnp.float32),
            pltpu.VMEM((D, LANES), jnp.float32),
            pltpu.SemaphoreType.DMA,
            pltpu.SemaphoreType.DMA,
            pltpu.SemaphoreType.DMA,
            pltpu.SemaphoreType.DMA,
            pltpu.SemaphoreType.DMA,
        ],
    )
    def lookup_kernel(tab_t, x_t, bias_s, out_t,
                      row_v, xcol_v, orow_v0, orow_v1, bias_v,
                      sem, g0, g1, s0, s1):
        wid = lax.axis_index("s") * nc + lax.axis_index("c")
        u0 = wid * per_w
        orows = (orow_v0, orow_v1)
        ssem = (s0, s1)

        f0 = u0 // D
        pltpu.async_copy(x_t.at[f0], xcol_v, sem).wait()
        pltpu.async_copy(bias_s.at[f0], bias_v, sem).wait()
        pltpu.async_copy(tab_t.at[u0 // D, u0 % D], row_v, g0)

        def unit(i, _):
            for b in range(2):
                u = u0 + i + b
                f = u // D
                d = u % D
                @pl.when(jnp.logical_and(i + b > 0, d == 0))
                def _():
                    pltpu.async_copy(x_t.at[f], xcol_v, sem).wait()
                    pltpu.async_copy(bias_s.at[f], bias_v, sem).wait()

                bvec = bias_v[d]
                pltpu.make_async_copy(tab_t.at[f, d], row_v, g0).wait()

                @pl.when(i + b >= 2)
                def _():
                    pltpu.make_async_copy(orows[b], out_t.at[f, d],
                                          ssem[b]).wait()

                @plsc.parallel_loop(0, n_vec, unroll=8)
                def _gather16(k):
                    iv = xcol_v[pl.ds(k * LANES, LANES)]
                    vals = plsc.load_gather(row_v, [iv])
                    orows[b][pl.ds(k * LANES, LANES)] = vals + bvec

                pltpu.async_copy(orows[b], out_t.at[f, d], ssem[b])

                @pl.when(i + b + 1 < per_w)
                def _():
                    un = u + 1
                    pltpu.async_copy(tab_t.at[un // D, un % D], row_v, g0)
            return ()

        lax.fori_loop(0, per_w // 2, lambda i, c: unit(i * 2, c), ())
        for b in range(2):
            ul = u0 + per_w - 2 + b
            pltpu.make_async_copy(orows[b], out_t.at[ul // D, ul % D],
                                  ssem[b]).wait()

    return lookup_kernel, nw


def kernel(x, tables, bias):
    batch = x.shape[0]
    lookup_kernel, _ = _build(batch)
    tab_t = jnp.swapaxes(tables, 1, 2)
    x_t = jnp.swapaxes(x, 0, 1).astype(jnp.int32)
    bias_s = jnp.broadcast_to(bias[:, :, None], (N_FIELDS, D, LANES))
    out_t = lookup_kernel(tab_t, x_t, bias_s)
    return jnp.transpose(out_t, (2, 0, 1))


# speedup vs baseline: 5.7299x; 1.0058x over previous
import functools

import jax
import jax.numpy as jnp
from jax import lax
from jax.experimental import pallas as pl
from jax.experimental.pallas import tpu as pltpu
from jax.experimental.pallas import tpu_sc as plsc

N_FIELDS = 26
D = 64
VOCAB = 100000
LANES = 16


@functools.lru_cache(maxsize=None)
def _build(batch: int):
    info = plsc.get_sparse_core_info()
    nc, ns = info.num_cores, info.num_subcores
    nw = nc * ns
    units = N_FIELDS * D
    per_w = units // nw
    assert per_w * nw == units
    n_vec = batch // LANES
    assert n_vec * LANES == batch

    mesh = plsc.VectorSubcoreMesh(core_axis_name="c", subcore_axis_name="s")

    @functools.partial(
        pl.kernel,
        mesh=mesh,
        compiler_params=pltpu.CompilerParams(needs_layout_passes=False),
        out_type=jax.ShapeDtypeStruct((N_FIELDS, D, batch), jnp.float32),
        scratch_types=[
            pltpu.VMEM((VOCAB,), jnp.float32),
            pltpu.VMEM((batch,), jnp.int32),
            pltpu.VMEM((batch,), jnp.float32),
            pltpu.VMEM((batch,), jnp.float32),
            pltpu.VMEM((D, LANES), jnp.float32),
            pltpu.SemaphoreType.DMA,
            pltpu.SemaphoreType.DMA,
            pltpu.SemaphoreType.DMA,
            pltpu.SemaphoreType.DMA,
            pltpu.SemaphoreType.DMA,
        ],
    )
    def lookup_kernel(tab_t, x_t, bias_s, out_t,
                      row_v, xcol_v, orow_v0, orow_v1, bias_v,
                      sem, g0, g1, s0, s1):
        wid = lax.axis_index("s") * nc + lax.axis_index("c")
        u0 = wid * per_w
        orows = (orow_v0, orow_v1)
        ssem = (s0, s1)

        f0 = u0 // D
        pltpu.async_copy(x_t.at[f0], xcol_v, sem).wait()
        pltpu.async_copy(bias_s.at[f0], bias_v, sem).wait()
        pltpu.async_copy(tab_t.at[u0 // D, u0 % D], row_v, g0)

        def unit(i, _):
            for b in range(2):
                u = u0 + i + b
                f = u // D
                d = u % D
                @pl.when(jnp.logical_and(i + b > 0, d == 0))
                def _():
                    pltpu.async_copy(x_t.at[f], xcol_v, sem).wait()
                    pltpu.async_copy(bias_s.at[f], bias_v, sem).wait()

                bvec = bias_v[d]
                pltpu.make_async_copy(tab_t.at[f, d], row_v, g0).wait()

                @pl.when(i + b >= 2)
                def _():
                    pltpu.make_async_copy(orows[b], out_t.at[f, d],
                                          ssem[b]).wait()

                @plsc.parallel_loop(0, n_vec, unroll=16)
                def _gather16(k):
                    iv = xcol_v[pl.ds(k * LANES, LANES)]
                    vals = plsc.load_gather(row_v, [iv])
                    orows[b][pl.ds(k * LANES, LANES)] = vals + bvec

                pltpu.async_copy(orows[b], out_t.at[f, d], ssem[b])

                @pl.when(i + b + 1 < per_w)
                def _():
                    un = u + 1
                    pltpu.async_copy(tab_t.at[un // D, un % D], row_v, g0)
            return ()

        lax.fori_loop(0, per_w // 2, lambda i, c: unit(i * 2, c), ())
        for b in range(2):
            ul = u0 + per_w - 2 + b
            pltpu.make_async_copy(orows[b], out_t.at[ul // D, ul % D],
                                  ssem[b]).wait()

    return lookup_kernel, nw


def kernel(x, tables, bias):
    batch = x.shape[0]
    lookup_kernel, _ = _build(batch)
    tab_t = jnp.swapaxes(tables, 1, 2)
    x_t = jnp.swapaxes(x, 0, 1).astype(jnp.int32)
    bias_s = jnp.broadcast_to(bias[:, :, None], (N_FIELDS, D, LANES))
    out_t = lookup_kernel(tab_t, x_t, bias_s)
    return jnp.transpose(out_t, (2, 0, 1))
